# baseline (speedup 1.0000x reference)
import functools

import jax
import jax.numpy as jnp
from jax import lax
from jax.experimental import pallas as pl
from jax.experimental.pallas import tpu as pltpu
from jax.experimental.pallas import tpu_sc as plsc

PAD = 0
EOS = 2
NUM_POSITIONS = 8
LANES = 16


@functools.lru_cache(maxsize=None)
def _build(batch: int, seq: int):
    info = plsc.get_sparse_core_info()
    num_workers = info.num_cores * info.num_subcores
    total = batch * seq
    chunk = total // num_workers
    assert total % num_workers == 0
    assert chunk % LANES == 0
    assert seq % chunk == 0
    steps = chunk // LANES

    mesh = plsc.VectorSubcoreMesh(core_axis_name="c", subcore_axis_name="s")

    @functools.partial(
        pl.kernel,
        mesh=mesh,
        out_type=[
            jax.ShapeDtypeStruct((total,), jnp.int32),
            jax.ShapeDtypeStruct((total,), jnp.int32),
        ],
        scratch_types=[
            pltpu.VMEM((chunk,), jnp.int32),
            pltpu.VMEM((chunk,), jnp.int32),
            pltpu.VMEM((chunk,), jnp.int32),
        ],
    )
    def run(x_hbm, pos_hbm, beh_hbm, xv, posv, behv):
        wid = lax.axis_index("s") * info.num_cores + lax.axis_index("c")
        base = wid * chunk
        pltpu.sync_copy(x_hbm.at[pl.ds(base, chunk)], xv)

        col0 = base % seq
        iota = lax.iota(jnp.int32, LANES)
        tok_idx = 1 + (iota & 8)
        lane_is_token = (iota & 7) == 1

        for k in range(steps):
            o = k * LANES
            v = xv[pl.ds(o, LANES)]
            mask = (v == PAD) | (v == EOS)
            colv = (col0 + o) + iota
            posp = ((colv - 1) & (NUM_POSITIONS - 1)) + 1
            pos = jnp.where(mask | (colv == 0), 0, posp)
            tok = v.at[tok_idx].get(mode="promise_in_bounds")
            mapped = jnp.where((tok >= 10) & (tok <= 13), tok - 9, tok)
            beh = jnp.where(mask | lane_is_token, 0, mapped)
            posv[pl.ds(o, LANES)] = pos
            behv[pl.ds(o, LANES)] = beh

        pltpu.sync_copy(posv, pos_hbm.at[pl.ds(base, chunk)])
        pltpu.sync_copy(behv, beh_hbm.at[pl.ds(base, chunk)])

    return run


def kernel(input_id_sequence):
    batch, seq = input_id_sequence.shape
    run = _build(batch, seq)
    pos, beh = run(input_id_sequence.reshape(-1))
    return pos.reshape(batch, seq), beh.reshape(batch, seq)


# speedup vs baseline: 1.1176x; 1.1176x over previous
import functools

import jax
import jax.numpy as jnp
from jax import lax
from jax.experimental import pallas as pl
from jax.experimental.pallas import tpu as pltpu
from jax.experimental.pallas import tpu_sc as plsc

PAD = 0
EOS = 2
NUM_POSITIONS = 8
LANES = 16
UNROLL = 4


@functools.lru_cache(maxsize=None)
def _build(batch: int, seq: int):
    info = plsc.get_sparse_core_info()
    num_workers = info.num_cores * info.num_subcores
    total = batch * seq
    chunk = total // num_workers
    assert total % num_workers == 0
    assert chunk % (LANES * UNROLL) == 0
    assert seq % chunk == 0
    per_row = seq // chunk
    steps = chunk // (LANES * UNROLL)

    mesh = plsc.VectorSubcoreMesh(core_axis_name="c", subcore_axis_name="s")

    @functools.partial(
        pl.kernel,
        mesh=mesh,
        out_type=[
            jax.ShapeDtypeStruct((batch, seq), jnp.int32),
            jax.ShapeDtypeStruct((batch, seq), jnp.int32),
        ],
        scratch_types=[
            pltpu.VMEM((chunk,), jnp.int32),
            pltpu.VMEM((chunk,), jnp.int32),
            pltpu.VMEM((chunk,), jnp.int32),
        ],
    )
    def run(x_hbm, pos_hbm, beh_hbm, xv, posv, behv):
        wid = lax.axis_index("s") * info.num_cores + lax.axis_index("c")
        row = wid // per_row
        col0 = (wid % per_row) * chunk
        pltpu.sync_copy(x_hbm.at[row, pl.ds(col0, chunk)], xv)

        iota = lax.iota(jnp.int32, LANES)
        tok_idx = 1 + (iota & 8)
        lane_is_token = (iota & 7) == 1

        def step(i, _):
            for u in range(UNROLL):
                o = i * (LANES * UNROLL) + u * LANES
                v = xv[pl.ds(o, LANES)]
                mask = (v == PAD) | (v == EOS)
                colv = (col0 + o) + iota
                posp = ((colv - 1) & (NUM_POSITIONS - 1)) + 1
                pos = jnp.where(mask | (colv == 0), 0, posp)
                tok = v.at[tok_idx].get(mode="promise_in_bounds")
                mapped = jnp.where((tok >= 10) & (tok <= 13), tok - 9, tok)
                beh = jnp.where(mask | lane_is_token, 0, mapped)
                posv[pl.ds(o, LANES)] = pos
                behv[pl.ds(o, LANES)] = beh
            return 0

        lax.fori_loop(0, steps, step, 0, unroll=False)

        pltpu.sync_copy(posv, pos_hbm.at[row, pl.ds(col0, chunk)])
        pltpu.sync_copy(behv, beh_hbm.at[row, pl.ds(col0, chunk)])

    return run


def kernel(input_id_sequence):
    batch, seq = input_id_sequence.shape
    run = _build(batch, seq)
    pos, beh = run(input_id_sequence)
    return pos, beh


# speedup vs baseline: 1.1951x; 1.0693x over previous
import functools

import jax
import jax.numpy as jnp
from jax import lax
from jax.experimental import pallas as pl
from jax.experimental.pallas import tpu as pltpu
from jax.experimental.pallas import tpu_sc as plsc

PAD = 0
EOS = 2
NUM_POSITIONS = 8
LANES = 16
UNROLL = 4


@functools.lru_cache(maxsize=None)
def _build(batch: int, seq: int):
    info = plsc.get_sparse_core_info()
    num_cores = 1
    num_workers = num_cores * info.num_subcores
    total = batch * seq
    chunk = total // num_workers
    assert total % num_workers == 0
    assert chunk % (LANES * UNROLL) == 0
    assert seq % chunk == 0
    per_row = seq // chunk
    steps = chunk // (LANES * UNROLL)

    mesh = plsc.VectorSubcoreMesh(
        core_axis_name="c", subcore_axis_name="s", num_cores=num_cores
    )

    @functools.partial(
        pl.kernel,
        mesh=mesh,
        out_type=[
            jax.ShapeDtypeStruct((batch, seq), jnp.int32),
            jax.ShapeDtypeStruct((batch, seq), jnp.int32),
        ],
        scratch_types=[
            pltpu.VMEM((chunk,), jnp.int32),
            pltpu.VMEM((chunk,), jnp.int32),
            pltpu.VMEM((chunk,), jnp.int32),
        ],
    )
    def run(x_hbm, pos_hbm, beh_hbm, xv, posv, behv):
        wid = lax.axis_index("s") * num_cores + lax.axis_index("c")
        row = wid // per_row
        col0 = (wid % per_row) * chunk
        pltpu.sync_copy(x_hbm.at[row, pl.ds(col0, chunk)], xv)

        iota = lax.iota(jnp.int32, LANES)
        tok_idx = 1 + (iota & 8)
        lane_is_token = (iota & 7) == 1

        def step(i, _):
            for u in range(UNROLL):
                o = i * (LANES * UNROLL) + u * LANES
                v = xv[pl.ds(o, LANES)]
                mask = (v == PAD) | (v == EOS)
                colv = (col0 + o) + iota
                posp = ((colv - 1) & (NUM_POSITIONS - 1)) + 1
                pos = jnp.where(mask | (colv == 0), 0, posp)
                tok = v.at[tok_idx].get(mode="promise_in_bounds")
                mapped = jnp.where((tok >= 10) & (tok <= 13), tok - 9, tok)
                beh = jnp.where(mask | lane_is_token, 0, mapped)
                posv[pl.ds(o, LANES)] = pos
                behv[pl.ds(o, LANES)] = beh
            return 0

        lax.fori_loop(0, steps, step, 0, unroll=False)

        pltpu.sync_copy(posv, pos_hbm.at[row, pl.ds(col0, chunk)])
        pltpu.sync_copy(behv, beh_hbm.at[row, pl.ds(col0, chunk)])

    return run


def kernel(input_id_sequence):
    batch, seq = input_id_sequence.shape
    run = _build(batch, seq)
    pos, beh = run(input_id_sequence)
    return pos, beh
